# Initial kernel scaffold; baseline (speedup 1.0000x reference)
import jax
import jax.numpy as jnp
from jax.experimental import pallas as pl


def kernel(features, rois):
    raise NotImplementedError("write your pallas kernel here")


# speedup vs baseline: 16.0375x; 16.0375x over previous
import jax
import jax.numpy as jnp
from jax.experimental import pallas as pl
from jax.experimental.pallas import tpu as pltpu

_OUT = 7


def _roi_kernel(rois_ref, order_ref, fmap_ref, out_ref, wred_ref):
    i = pl.program_id(0)
    W = fmap_ref.shape[1]
    H = fmap_ref.shape[2]
    C = fmap_ref.shape[3]
    x1 = rois_ref[i, 1]
    y1 = rois_ref[i, 2]
    x2 = rois_ref[i, 3]
    y2 = rois_ref[i, 4]
    roi_w = x2 - x1 + 1
    roi_h = y2 - y1 + 1

    for w in range(_OUT):
        ws = (w * roi_w) // _OUT
        we = ((w + 1) * roi_w) // _OUT
        cs = x1 + ws
        ce = x1 + we
        sw = jnp.minimum(cs, W - 5)
        chunk = fmap_ref[0, pl.ds(sw, 5), :, :]
        cid = jax.lax.broadcasted_iota(jnp.int32, (5, H, C), 0) + sw
        valid = (cid >= cs) & (cid < ce)
        masked = jnp.where(valid, chunk, -jnp.inf)
        wred_ref[w] = jnp.max(masked, axis=0)

    wb = jax.lax.broadcasted_iota(jnp.int32, (8, C), 0)
    ws_v = (wb * roi_w) // _OUT
    we_v = ((wb + 1) * roi_w) // _OUT
    ne_w = we_v > ws_v

    for h in range(_OUT):
        hs = (h * roi_h) // _OUT
        he = ((h + 1) * roi_h) // _OUT
        rs = y1 + hs
        re = y1 + he
        sh = jnp.minimum((rs // 8) * 8, H - 16)
        sh = pl.multiple_of(sh, 8)
        rows = wred_ref[:, pl.ds(sh, 16), :]
        rid = jax.lax.broadcasted_iota(jnp.int32, (8, 16, C), 1) + sh
        valid = (rid >= rs) & (rid < re)
        masked = jnp.where(valid, rows, -jnp.inf)
        mx = jnp.max(masked, axis=1)
        ne = ne_w & (he > hs)
        out_ref[0, h] = jnp.where(ne, mx, 0.0)[:_OUT]


def kernel(features, rois):
    B, C, H, W = features.shape
    N = rois.shape[0]
    feats = jnp.transpose(features, (0, 3, 2, 1))
    rois32 = rois.astype(jnp.int32)
    order = jnp.argsort(rois32[:, 0]).astype(jnp.int32)
    rois_s = rois32[order]

    grid_spec = pltpu.PrefetchScalarGridSpec(
        num_scalar_prefetch=2,
        grid=(N,),
        in_specs=[
            pl.BlockSpec((1, W, H, C), lambda i, r, o: (r[i, 0], 0, 0, 0)),
        ],
        out_specs=pl.BlockSpec((1, _OUT, _OUT, C), lambda i, r, o: (o[i], 0, 0, 0)),
        scratch_shapes=[pltpu.VMEM((8, H, C), jnp.float32)],
    )
    out = pl.pallas_call(
        _roi_kernel,
        grid_spec=grid_spec,
        out_shape=jax.ShapeDtypeStruct((N, _OUT, _OUT, C), jnp.float32),
    )(rois_s, order, feats)
    return jnp.transpose(out, (0, 3, 1, 2))


# speedup vs baseline: 19.0722x; 1.1892x over previous
import functools

import jax
import jax.numpy as jnp
from jax import lax
from jax.experimental import pallas as pl
from jax.experimental.pallas import tpu as pltpu
from jax.experimental.pallas import tpu_sc as plsc

_OUT = 7
_NBINS = _OUT * _OUT


def _make_sc_call(N, C, H, W):
    info = plsc.get_sparse_core_info()
    NC, NS = info.num_cores, info.num_subcores
    NW = NC * NS
    assert N % NW == 0
    R = N // NW
    nck = C // 16

    mesh = plsc.VectorSubcoreMesh(core_axis_name="c", subcore_axis_name="s")

    @functools.partial(
        pl.kernel,
        mesh=mesh,
        out_type=jax.ShapeDtypeStruct((N * 56, C), jnp.float32),
        scratch_types=[
            pltpu.VMEM((R * 16,), jnp.int32),
            pltpu.VMEM((8, W, C), jnp.float32),
            pltpu.VMEM((56, C), jnp.float32),
            pltpu.SemaphoreType.DMA,
        ],
    )
    def body(feats_hbm, rois_hbm, out_hbm, rois_v, buf, obuf, sem):
        wid = lax.axis_index("s") * NC + lax.axis_index("c")
        base = wid * R
        pltpu.sync_copy(rois_hbm.at[pl.ds(base * 16, R * 16)], rois_v)

        def roi_body(r, _):
            v = rois_v[pl.ds(r * 16, 16)]
            b = v[0]
            x1 = v[1]
            y1 = v[2]
            x2 = v[3]
            y2 = v[4]
            roi_w = x2 - x1 + 1
            roi_h = y2 - y1 + 1

            def hbin_body(h, _):
                rs = y1 + (h * roi_h) // _OUT
                re = y1 + ((h + 1) * roi_h) // _OUT
                bh = re - rs

                def dma_issue(j, _):
                    pltpu.async_copy(
                        feats_hbm.at[pl.ds(((b * H) + rs + j) * W, W)],
                        buf.at[j],
                        sem,
                    )
                    return 0

                lax.fori_loop(0, bh, dma_issue, 0)

                def dma_drain(j, _):
                    pltpu.make_async_copy(
                        feats_hbm.at[pl.ds(0, W)], buf.at[j], sem
                    ).wait()
                    return 0

                lax.fori_loop(0, bh, dma_drain, 0)

                def wbin_body(w, _):
                    ws = (w * roi_w) // _OUT
                    we = ((w + 1) * roi_w) // _OUT
                    bw = we - ws
                    cs = x1 + ws

                    init = tuple(
                        jnp.full((16,), -jnp.inf, jnp.float32) for _ in range(nck)
                    )

                    def row_body(jr, acc):
                        def col_body(t, acc2):
                            col = cs + t
                            return tuple(
                                jnp.maximum(
                                    acc2[k], buf[jr, col, pl.ds(k * 16, 16)]
                                )
                                for k in range(nck)
                            )

                        return lax.fori_loop(0, bw, col_body, acc)

                    acc = lax.fori_loop(0, bh, row_body, init)
                    ne = (bh > 0) & (bw > 0)
                    bin_i = h * _OUT + w
                    for k in range(nck):
                        obuf[bin_i, pl.ds(k * 16, 16)] = jnp.where(
                            ne, acc[k], 0.0
                        )
                    return 0

                lax.fori_loop(0, _OUT, wbin_body, 0)
                return 0

            lax.fori_loop(0, _OUT, hbin_body, 0)
            pltpu.sync_copy(obuf, out_hbm.at[pl.ds((base + r) * 56, 56)])
            return 0

        lax.fori_loop(0, R, roi_body, 0)

    return body


def kernel(features, rois):
    B, C, H, W = features.shape
    N = rois.shape[0]
    feats = jnp.transpose(features, (0, 2, 3, 1)).reshape(B * H * W, C)
    roisp = jnp.zeros((N, 16), jnp.int32).at[:, :5].set(rois.astype(jnp.int32))
    roisp = roisp.reshape(N * 16)
    out = _make_sc_call(N, C, H, W)(feats, roisp)
    out = out.reshape(N, 56, C)[:, :_NBINS]
    return out.transpose(0, 2, 1).reshape(N, C, _OUT, _OUT)


# speedup vs baseline: 27.6316x; 1.4488x over previous
import functools

import jax
import jax.numpy as jnp
from jax import lax
from jax.experimental import pallas as pl
from jax.experimental.pallas import tpu as pltpu
from jax.experimental.pallas import tpu_sc as plsc

_OUT = 7
_NBINS = _OUT * _OUT
_OSTRIDE = 56


def _make_sc_call(N, C, H, W):
    info = plsc.get_sparse_core_info()
    NC, NS = info.num_cores, info.num_subcores
    NW = NC * NS
    assert N % NW == 0
    R = N // NW
    assert R >= 2 and (R * _OUT) % 2 == 0
    NT = R * _OUT
    nck = C // 16
    obytes = _OSTRIDE * C * 4

    mesh = plsc.VectorSubcoreMesh(core_axis_name="c", subcore_axis_name="s")

    @functools.partial(
        pl.kernel,
        mesh=mesh,
        out_type=jax.ShapeDtypeStruct((N * _OSTRIDE, C), jnp.float32),
        scratch_types=[
            pltpu.VMEM((R * 16,), jnp.int32),
            pltpu.VMEM((5, W, C), jnp.float32),
            pltpu.VMEM((5, W, C), jnp.float32),
            pltpu.VMEM((2 * _OSTRIDE, C), jnp.float32),
            pltpu.SemaphoreType.DMA,
            pltpu.SemaphoreType.DMA,
            pltpu.SemaphoreType.DMA,
        ],
    )
    def body(feats_hbm, rois_hbm, out_hbm, rois_v, buf0, buf1, obuf, sem0, sem1, semo):
        wid = lax.axis_index("s") * NC + lax.axis_index("c")
        base = wid * R
        pltpu.sync_copy(rois_hbm.at[pl.ds(base * 16, R * 16)], rois_v)

        def task_params(t):
            r = t // _OUT
            h = t - r * _OUT
            v = rois_v[pl.ds(r * 16, 16)]
            b = v[0]
            x1 = v[1]
            y1 = v[2]
            x2 = v[3]
            y2 = v[4]
            roi_w = x2 - x1 + 1
            roi_h = y2 - y1 + 1
            rs = y1 + (h * roi_h) // _OUT
            re = y1 + ((h + 1) * roi_h) // _OUT
            return r, h, b, x1, roi_w, rs, re

        def issue(t, buf, sem):
            @pl.when(t < NT)
            def _():
                r, h, b, x1, roi_w, rs, re = task_params(t)

                def dma_issue(j, _):
                    pltpu.async_copy(
                        feats_hbm.at[pl.ds(((b * H) + rs + j) * W, W)],
                        buf.at[j],
                        sem,
                    )
                    return 0

                lax.fori_loop(0, re - rs, dma_issue, 0)

        def consume(t, buf, sem):
            r, h, b, x1, roi_w, rs, re = task_params(t)
            bh = re - rs
            oslot = (r % 2) * _OSTRIDE

            @pl.when((h == 0) & (r >= 2))
            def _():
                pltpu.make_async_copy(
                    obuf.at[pl.ds(0, _OSTRIDE)],
                    out_hbm.at[pl.ds(0, _OSTRIDE)],
                    semo,
                ).wait()

            def dma_drain(j, _):
                pltpu.make_async_copy(
                    feats_hbm.at[pl.ds(0, W)], buf.at[j], sem
                ).wait()
                return 0

            lax.fori_loop(0, bh, dma_drain, 0)

            def wbin_body(w, _):
                ws = (w * roi_w) // _OUT
                we = ((w + 1) * roi_w) // _OUT
                bw = we - ws
                cs = x1 + ws

                init = tuple(
                    jnp.full((16,), -jnp.inf, jnp.float32) for _ in range(nck)
                )

                def row_body(jr, acc):
                    def col_body(tt, acc2):
                        col = cs + tt
                        return tuple(
                            jnp.maximum(acc2[k], buf[jr, col, pl.ds(k * 16, 16)])
                            for k in range(nck)
                        )

                    return lax.fori_loop(0, bw, col_body, acc)

                acc = lax.fori_loop(0, bh, row_body, init)
                ne = (bh > 0) & (bw > 0)
                bin_i = oslot + h * _OUT + w
                for k in range(nck):
                    obuf[bin_i, pl.ds(k * 16, 16)] = jnp.where(ne, acc[k], 0.0)
                return 0

            lax.fori_loop(0, _OUT, wbin_body, 0)

            @pl.when(h == _OUT - 1)
            def _():
                pltpu.async_copy(
                    obuf.at[pl.ds(oslot, _OSTRIDE)],
                    out_hbm.at[pl.ds((base + r) * _OSTRIDE, _OSTRIDE)],
                    semo,
                )

        issue(0, buf0, sem0)

        def k_body(k, _):
            t = 2 * k
            issue(t + 1, buf1, sem1)
            consume(t, buf0, sem0)
            issue(t + 2, buf0, sem0)
            consume(t + 1, buf1, sem1)
            return 0

        lax.fori_loop(0, NT // 2, k_body, 0)

        for _ in range(2):
            pltpu.make_async_copy(
                obuf.at[pl.ds(0, _OSTRIDE)],
                out_hbm.at[pl.ds(0, _OSTRIDE)],
                semo,
            ).wait()

    return body


def kernel(features, rois):
    B, C, H, W = features.shape
    N = rois.shape[0]
    feats = jnp.transpose(features, (0, 2, 3, 1)).reshape(B * H * W, C)
    roisp = jnp.zeros((N, 16), jnp.int32).at[:, :5].set(rois.astype(jnp.int32))
    roisp = roisp.reshape(N * 16)
    out = _make_sc_call(N, C, H, W)(feats, roisp)
    out = out.reshape(N, _OSTRIDE, C)[:, :_NBINS]
    return out.transpose(0, 2, 1).reshape(N, C, _OUT, _OUT)
